# baseline (speedup 1.0000x reference)
import functools

import jax
import jax.numpy as jnp
from jax import lax
from jax.experimental import pallas as pl
from jax.experimental.pallas import tpu as pltpu
from jax.experimental.pallas import tpu_sc as plsc

_HID = 128
_NW = 32
_C = 64
_DEPTH = 4
_MLEN = 25

_BR_B = 1024
_BR_A = 3200


def _make_gather_sum(bp):
    cpw = bp // _NW
    nchunk = cpw // _C
    mesh = plsc.VectorSubcoreMesh(core_axis_name="c", subcore_axis_name="s")

    @functools.partial(
        pl.kernel,
        mesh=mesh,
        out_type=jax.ShapeDtypeStruct((bp, _HID), jnp.float32),
        scratch_types=[
            pltpu.VMEM((2, 6, _C), jnp.int32),
            pltpu.VMEM((2, 6, _C, _HID), jnp.float32),
            pltpu.VMEM((2, _C, _HID), jnp.float32),
            pltpu.SemaphoreType.DMA,
            pltpu.SemaphoreType.DMA,
        ],
    )
    def gather_sum(table_hbm, idx_hbm, out_hbm, idx_v, bufs_v, out_v,
                   sem_a, sem_b):
        cid = lax.axis_index("c")
        sid = lax.axis_index("s")
        wid = sid * 2 + cid
        base = wid * cpw
        sems = (sem_a, sem_b)

        def fire(ci, s):
            pltpu.sync_copy(idx_hbm.at[wid, ci], idx_v.at[s])
            for j in range(6):
                pltpu.async_copy(table_hbm.at[idx_v.at[s, j]],
                                 bufs_v.at[s, j], sems[s])

        def drain(s):
            for j in range(6):
                pltpu.make_async_copy(table_hbm.at[idx_v.at[s, j]],
                                      bufs_v.at[s, j], sems[s]).wait()

        def add(s):
            def row(r, c2):
                for l in range(8):
                    sl = pl.ds(l * 16, 16)
                    acc = bufs_v[s, 0, r, sl] + bufs_v[s, 1, r, sl]
                    acc = acc + bufs_v[s, 2, r, sl]
                    acc = acc + bufs_v[s, 3, r, sl]
                    acc = acc + bufs_v[s, 4, r, sl]
                    acc = acc + bufs_v[s, 5, r, sl]
                    out_v[s, r, sl] = acc
                return c2

            lax.fori_loop(0, _C, row, 0)

        fire(0, 0)
        fire(1, 1)

        def pair(p, carry):
            c0 = 2 * p
            for s in range(2):
                drain(s)
                add(s)

                @pl.when(c0 + 2 + s < nchunk)
                def _():
                    fire(c0 + 2 + s, s)

                pltpu.sync_copy(out_v.at[s],
                                out_hbm.at[pl.ds(base + (c0 + s) * _C, _C)])
            return carry

        lax.fori_loop(0, nchunk // 2, pair, 0)

    return gather_sum


def _prep_idx(g, bp):
    b = g.shape[0]
    cpw = bp // _NW
    nchunk = cpw // _C
    gi = g.astype(jnp.int32)
    gi = jnp.pad(gi, ((0, bp - b), (0, 0)))
    gi = gi.reshape(_NW, nchunk, _C, 6)
    return gi.transpose(0, 1, 3, 2)


def _nt(x, w):
    return lax.dot_general(x, w, (((1,), (1,)), ((), ())),
                           preferred_element_type=jnp.float32)


def _bonds_input_body(x_ref, w_ref, bin_ref, msg_ref):
    b = _nt(x_ref[...], w_ref[...])
    bin_ref[...] = b
    msg_ref[...] = jnp.maximum(b, 0.0)


def _bonds_input(fbonds, w_i, bp):
    n, k = fbonds.shape
    grid = (n + _BR_B - 1) // _BR_B
    return pl.pallas_call(
        _bonds_input_body,
        grid=(grid,),
        in_specs=[
            pl.BlockSpec((_BR_B, k), lambda i: (i, 0)),
            pl.BlockSpec((_HID, k), lambda i: (0, 0)),
        ],
        out_specs=[
            pl.BlockSpec((_BR_B, _HID), lambda i: (i, 0)),
            pl.BlockSpec((_BR_B, _HID), lambda i: (i, 0)),
        ],
        out_shape=[
            jax.ShapeDtypeStruct((bp, _HID), jnp.float32),
            jax.ShapeDtypeStruct((bp, _HID), jnp.float32),
        ],
    )(fbonds, w_i)


def _iter_body(nei_ref, bin_ref, w_ref, out_ref):
    h = _nt(nei_ref[...], w_ref[...])
    out_ref[...] = jnp.maximum(bin_ref[...] + h, 0.0)


def _iter_half_body(nei_ref, bin_ref, w_ref, alias_ref, out_ref):
    del alias_ref
    h = _nt(nei_ref[...], w_ref[...])
    out_ref[...] = jnp.maximum(bin_ref[...] + h, 0.0)


def _iter_step_half(nei_half, binput, w_h, bp, blk_off, msg_alias=None):
    grid = nei_half.shape[0] // _BR_B
    if msg_alias is None:
        return pl.pallas_call(
            _iter_body,
            grid=(grid,),
            in_specs=[
                pl.BlockSpec((_BR_B, _HID), lambda i: (i, 0)),
                pl.BlockSpec((_BR_B, _HID), lambda i, b=blk_off: (i + b, 0)),
                pl.BlockSpec((_HID, _HID), lambda i: (0, 0)),
            ],
            out_specs=pl.BlockSpec((_BR_B, _HID),
                                   lambda i, b=blk_off: (i + b, 0)),
            out_shape=jax.ShapeDtypeStruct((bp, _HID), jnp.float32),
        )(nei_half, binput, w_h)
    return pl.pallas_call(
        _iter_half_body,
        grid=(grid,),
        in_specs=[
            pl.BlockSpec((_BR_B, _HID), lambda i: (i, 0)),
            pl.BlockSpec((_BR_B, _HID), lambda i, b=blk_off: (i + b, 0)),
            pl.BlockSpec((_HID, _HID), lambda i: (0, 0)),
            pl.BlockSpec((8, _HID), lambda i: (0, 0)),
        ],
        out_specs=pl.BlockSpec((_BR_B, _HID),
                               lambda i, b=blk_off: (i + b, 0)),
        out_shape=jax.ShapeDtypeStruct((bp, _HID), jnp.float32),
        input_output_aliases={3: 0},
    )(nei_half, binput, w_h, msg_alias)


def _final_body(n_valid, fa_ref, nei_ref, wa_ref, wn_ref, bo_ref, out_ref):
    h = _nt(fa_ref[...], wa_ref[...]) + _nt(nei_ref[...], wn_ref[...])
    h = jnp.maximum(h + bo_ref[...], 0.0)
    row = (lax.broadcasted_iota(jnp.int32, (_BR_A, _HID), 0)
           + pl.program_id(0) * _BR_A)
    h = jnp.where(row < n_valid, h, 0.0)
    nm = _BR_A // _MLEN
    am = lax.broadcasted_iota(jnp.int32, (nm, _BR_A), 1) // _MLEN
    mm = lax.broadcasted_iota(jnp.int32, (nm, _BR_A), 0)
    pool = (am == mm).astype(jnp.float32)
    out_ref[...] = lax.dot_general(pool, h, (((1,), (0,)), ((), ())),
                                   preferred_element_type=jnp.float32)


def _final_stage(fatoms, nei_a, w_oa, w_on, b_o2d, ap):
    n, fa = fatoms.shape
    grid = ap // _BR_A
    nm = _BR_A // _MLEN
    return pl.pallas_call(
        functools.partial(_final_body, n),
        grid=(grid,),
        in_specs=[
            pl.BlockSpec((_BR_A, fa), lambda i: (i, 0)),
            pl.BlockSpec((_BR_A, _HID), lambda i: (i, 0)),
            pl.BlockSpec((_HID, fa), lambda i: (0, 0)),
            pl.BlockSpec((_HID, _HID), lambda i: (0, 0)),
            pl.BlockSpec((1, _HID), lambda i: (0, 0)),
        ],
        out_specs=pl.BlockSpec((nm, _HID), lambda i: (i, 0)),
        out_shape=jax.ShapeDtypeStruct((grid * nm, _HID), jnp.float32),
    )(fatoms, nei_a, w_oa, w_on, b_o2d)


def kernel(fatoms, fbonds, agraph, bgraph, scope, W_i, W_h, W_o, b_o):
    n_atoms, atom_fdim = fatoms.shape
    n_bonds = fbonds.shape[0]
    n_mols = scope.shape[0]

    bp = 204800
    hp = bp // 2
    ap = 102400

    bg0_idx = _prep_idx(bgraph[:hp], hp)
    bg1_idx = _prep_idx(bgraph[hp:], hp)
    ag_idx = _prep_idx(agraph, ap)
    hblk = hp // _BR_B

    binput, message = _bonds_input(fbonds, W_i, bp)
    gather_h = _make_gather_sum(hp)
    for _ in range(_DEPTH - 1):
        nei0 = gather_h(message, bg0_idx)
        nei1 = gather_h(message, bg1_idx)
        m0 = _iter_step_half(nei0, binput, W_h, bp, 0)
        message = _iter_step_half(nei1, binput, W_h, bp, hblk, msg_alias=m0)

    nei_a = gather_h(message, ag_idx)

    w_oa = W_o[:, :atom_fdim]
    w_on = W_o[:, atom_fdim:]
    mol_sums = _final_stage(fatoms, nei_a, w_oa, w_on,
                            b_o.reshape(1, _HID), ap)
    lengths = scope[:, 1].astype(jnp.float32)
    return mol_sums[:n_mols] / lengths[:, None]


# speedup vs baseline: 1.7636x; 1.7636x over previous
import functools

import jax
import jax.numpy as jnp
from jax import lax
from jax.experimental import pallas as pl
from jax.experimental.pallas import tpu as pltpu
from jax.experimental.pallas import tpu_sc as plsc

_HID = 128
_NW = 32
_C = 64
_DEPTH = 4
_MLEN = 25

_BR_B = 1024
_BR_A = 3200


def _make_gather_sum(bp, k0):
    psz = bp // 16
    ktot = psz // _C
    k1 = ktot - k0
    assert k0 % 2 == 0 and k1 % 2 == 0
    mesh = plsc.VectorSubcoreMesh(core_axis_name="c", subcore_axis_name="s")

    @functools.partial(
        pl.kernel,
        mesh=mesh,
        out_type=jax.ShapeDtypeStruct((bp, _HID), jnp.float32),
        scratch_types=[
            pltpu.VMEM((2, 6, _C), jnp.int32),
            pltpu.VMEM((2, 6, _C, _HID), jnp.float32),
            pltpu.VMEM((2, _C, _HID), jnp.float32),
            pltpu.SemaphoreType.DMA,
            pltpu.SemaphoreType.DMA,
        ],
    )
    def gather_sum(table_hbm, idx_hbm, out_hbm, idx_v, bufs_v, out_v,
                   sem_a, sem_b):
        cid = lax.axis_index("c")
        sid = lax.axis_index("s")
        gbase = sid * ktot + cid * k0
        nchunk = jnp.where(cid == 0, k0, k1)
        base = gbase * _C
        sems = (sem_a, sem_b)

        def fire(ci, s):
            pltpu.sync_copy(idx_hbm.at[gbase + ci], idx_v.at[s])
            for j in range(6):
                pltpu.async_copy(table_hbm.at[idx_v.at[s, j]],
                                 bufs_v.at[s, j], sems[s])

        def drain(s):
            for j in range(6):
                pltpu.make_async_copy(table_hbm.at[idx_v.at[s, j]],
                                      bufs_v.at[s, j], sems[s]).wait()

        def add(s):
            def row(r, c2):
                for l in range(8):
                    sl = pl.ds(l * 16, 16)
                    acc = bufs_v[s, 0, r, sl] + bufs_v[s, 1, r, sl]
                    acc = acc + bufs_v[s, 2, r, sl]
                    acc = acc + bufs_v[s, 3, r, sl]
                    acc = acc + bufs_v[s, 4, r, sl]
                    acc = acc + bufs_v[s, 5, r, sl]
                    out_v[s, r, sl] = acc
                return c2

            lax.fori_loop(0, _C, row, 0)

        fire(0, 0)
        fire(1, 1)

        def pair(p, carry):
            c0 = 2 * p
            for s in range(2):
                drain(s)
                add(s)

                @pl.when(c0 + 2 + s < nchunk)
                def _():
                    fire(c0 + 2 + s, s)

                pltpu.sync_copy(out_v.at[s],
                                out_hbm.at[pl.ds(base + (c0 + s) * _C, _C)])
            return carry

        lax.fori_loop(0, nchunk // 2, pair, 0)

    return gather_sum


def _prep_idx(g, bp):
    b = g.shape[0]
    gi = g.astype(jnp.int32)
    gi = jnp.pad(gi, ((0, bp - b), (0, 0)))
    gi = gi.reshape(bp // _C, _C, 6)
    return gi.transpose(0, 2, 1)


def _nt(x, w):
    return lax.dot_general(x, w, (((1,), (1,)), ((), ())),
                           preferred_element_type=jnp.float32)


def _bonds_input_body(x_ref, w_ref, bin_ref, msg_ref):
    b = _nt(x_ref[...], w_ref[...])
    bin_ref[...] = b
    msg_ref[...] = jnp.maximum(b, 0.0)


def _bonds_input(fbonds, w_i, bp):
    n, k = fbonds.shape
    grid = bp // _BR_B
    return pl.pallas_call(
        _bonds_input_body,
        grid=(grid,),
        in_specs=[
            pl.BlockSpec((_BR_B, k), lambda i: (i, 0)),
            pl.BlockSpec((_HID, k), lambda i: (0, 0)),
        ],
        out_specs=[
            pl.BlockSpec((_BR_B, _HID), lambda i: (i, 0)),
            pl.BlockSpec((_BR_B, _HID), lambda i: (i, 0)),
        ],
        out_shape=[
            jax.ShapeDtypeStruct((bp, _HID), jnp.float32),
            jax.ShapeDtypeStruct((bp, _HID), jnp.float32),
        ],
    )(fbonds, w_i)


def _iter_body(nei_ref, bin_ref, w_ref, out_ref):
    h = _nt(nei_ref[...], w_ref[...])
    out_ref[...] = jnp.maximum(bin_ref[...] + h, 0.0)


def _iter_step(nei, binput, w_h):
    bp = nei.shape[0]
    grid = bp // _BR_B
    return pl.pallas_call(
        _iter_body,
        grid=(grid,),
        in_specs=[
            pl.BlockSpec((_BR_B, _HID), lambda i: (i, 0)),
            pl.BlockSpec((_BR_B, _HID), lambda i: (i, 0)),
            pl.BlockSpec((_HID, _HID), lambda i: (0, 0)),
        ],
        out_specs=pl.BlockSpec((_BR_B, _HID), lambda i: (i, 0)),
        out_shape=jax.ShapeDtypeStruct((bp, _HID), jnp.float32),
    )(nei, binput, w_h)


def _final_body(n_valid, fa_ref, nei_ref, wa_ref, wn_ref, bo_ref, out_ref):
    h = _nt(fa_ref[...], wa_ref[...]) + _nt(nei_ref[...], wn_ref[...])
    h = jnp.maximum(h + bo_ref[...], 0.0)
    row = (lax.broadcasted_iota(jnp.int32, (_BR_A, _HID), 0)
           + pl.program_id(0) * _BR_A)
    h = jnp.where(row < n_valid, h, 0.0)
    nm = _BR_A // _MLEN
    am = lax.broadcasted_iota(jnp.int32, (nm, _BR_A), 1) // _MLEN
    mm = lax.broadcasted_iota(jnp.int32, (nm, _BR_A), 0)
    pool = (am == mm).astype(jnp.float32)
    out_ref[...] = lax.dot_general(pool, h, (((1,), (0,)), ((), ())),
                                   preferred_element_type=jnp.float32)


def _final_stage(fatoms, nei_a, w_oa, w_on, b_o2d, ap):
    n, fa = fatoms.shape
    grid = ap // _BR_A
    nm = _BR_A // _MLEN
    return pl.pallas_call(
        functools.partial(_final_body, n),
        grid=(grid,),
        in_specs=[
            pl.BlockSpec((_BR_A, fa), lambda i: (i, 0)),
            pl.BlockSpec((_BR_A, _HID), lambda i: (i, 0)),
            pl.BlockSpec((_HID, fa), lambda i: (0, 0)),
            pl.BlockSpec((_HID, _HID), lambda i: (0, 0)),
            pl.BlockSpec((1, _HID), lambda i: (0, 0)),
        ],
        out_specs=pl.BlockSpec((nm, _HID), lambda i: (i, 0)),
        out_shape=jax.ShapeDtypeStruct((grid * nm, _HID), jnp.float32),
    )(fatoms, nei_a, w_oa, w_on, b_o2d)


def kernel(fatoms, fbonds, agraph, bgraph, scope, W_i, W_h, W_o, b_o):
    n_atoms, atom_fdim = fatoms.shape
    n_bonds = fbonds.shape[0]
    n_mols = scope.shape[0]

    bp = 200704
    ap = 102400

    bg_idx = _prep_idx(bgraph, bp)
    ag_idx = _prep_idx(agraph, ap)

    binput, message = _bonds_input(fbonds, W_i, bp)
    gather_b = _make_gather_sum(bp, 82)
    for _ in range(_DEPTH - 1):
        nei = gather_b(message, bg_idx)
        message = _iter_step(nei, binput, W_h)

    gather_a = _make_gather_sum(ap, 42)
    nei_a = gather_a(message, ag_idx)

    w_oa = W_o[:, :atom_fdim]
    w_on = W_o[:, atom_fdim:]
    mol_sums = _final_stage(fatoms, nei_a, w_oa, w_on,
                            b_o.reshape(1, _HID), ap)
    lengths = scope[:, 1].astype(jnp.float32)
    return mol_sums[:n_mols] / lengths[:, None]


# speedup vs baseline: 1.8316x; 1.0385x over previous
import functools

import jax
import jax.numpy as jnp
from jax import lax
from jax.experimental import pallas as pl
from jax.experimental.pallas import tpu as pltpu
from jax.experimental.pallas import tpu_sc as plsc

_HID = 128
_NW = 32
_C = 64
_DEPTH = 4
_MLEN = 25

_BR_B = 1024
_BR_A = 3200


def _make_gather_sum(bp, k0):
    psz = bp // 16
    ktot = psz // _C
    k1 = ktot - k0
    assert k0 % 2 == 0 and k1 % 2 == 0
    mesh = plsc.VectorSubcoreMesh(core_axis_name="c", subcore_axis_name="s")

    @functools.partial(
        pl.kernel,
        mesh=mesh,
        out_type=jax.ShapeDtypeStruct((bp, _HID), jnp.float32),
        scratch_types=[
            pltpu.VMEM((2, 6, _C), jnp.int32),
            pltpu.VMEM((2, 6, _C, _HID), jnp.float32),
            pltpu.VMEM((2, _C, _HID), jnp.float32),
            pltpu.SemaphoreType.DMA,
            pltpu.SemaphoreType.DMA,
        ],
    )
    def gather_sum(table_hbm, idx_hbm, out_hbm, idx_v, bufs_v, out_v,
                   sem_a, sem_b):
        cid = lax.axis_index("c")
        sid = lax.axis_index("s")
        gbase = sid * ktot + cid * k0
        nchunk = jnp.where(cid == 0, k0, k1)
        base = gbase * _C
        sems = (sem_a, sem_b)

        def fire(ci, s):
            pltpu.sync_copy(idx_hbm.at[gbase + ci], idx_v.at[s])
            for j in range(6):
                pltpu.async_copy(table_hbm.at[idx_v.at[s, j]],
                                 bufs_v.at[s, j], sems[s])

        def drain(s):
            for j in range(6):
                pltpu.make_async_copy(table_hbm.at[idx_v.at[s, j]],
                                      bufs_v.at[s, j], sems[s]).wait()

        def add(s):
            def row(r, c2):
                for l in range(8):
                    sl = pl.ds(l * 16, 16)
                    acc = bufs_v[s, 0, r, sl] + bufs_v[s, 1, r, sl]
                    acc = acc + bufs_v[s, 2, r, sl]
                    acc = acc + bufs_v[s, 3, r, sl]
                    acc = acc + bufs_v[s, 4, r, sl]
                    acc = acc + bufs_v[s, 5, r, sl]
                    out_v[s, r, sl] = acc
                return c2

            lax.fori_loop(0, _C, row, 0)

        fire(0, 0)
        fire(1, 1)

        def pair(p, carry):
            c0 = 2 * p
            for s in range(2):
                drain(s)
                add(s)

                @pl.when(c0 + 2 + s < nchunk)
                def _():
                    fire(c0 + 2 + s, s)

                pltpu.sync_copy(out_v.at[s],
                                out_hbm.at[pl.ds(base + (c0 + s) * _C, _C)])
            return carry

        lax.fori_loop(0, nchunk // 2, pair, 0)

    return gather_sum


def _prep_idx(g, bp):
    b = g.shape[0]
    gi = g.astype(jnp.int32)
    gi = jnp.pad(gi, ((0, bp - b), (0, 0)))
    gi = gi.reshape(bp // _C, _C, 6)
    return gi.transpose(0, 2, 1)


def _nt(x, w):
    return lax.dot_general(x, w, (((1,), (1,)), ((), ())),
                           preferred_element_type=jnp.float32)


def _bonds_input_body(x_ref, w_ref, bin_ref, msg_ref):
    b = _nt(x_ref[...], w_ref[...])
    bin_ref[...] = b
    msg_ref[...] = jnp.maximum(b, 0.0)


def _bonds_input(fbonds, w_i, bp):
    n, k = fbonds.shape
    grid = bp // _BR_B
    return pl.pallas_call(
        _bonds_input_body,
        grid=(grid,),
        in_specs=[
            pl.BlockSpec((_BR_B, k), lambda i: (i, 0)),
            pl.BlockSpec((_HID, k), lambda i: (0, 0)),
        ],
        out_specs=[
            pl.BlockSpec((_BR_B, _HID), lambda i: (i, 0)),
            pl.BlockSpec((_BR_B, _HID), lambda i: (i, 0)),
        ],
        out_shape=[
            jax.ShapeDtypeStruct((bp, _HID), jnp.float32),
            jax.ShapeDtypeStruct((bp, _HID), jnp.float32),
        ],
    )(fbonds, w_i)


def _iter_body(nei_ref, bin_ref, w_ref, out_ref):
    h = _nt(nei_ref[...], w_ref[...])
    out_ref[...] = jnp.maximum(bin_ref[...] + h, 0.0)


def _iter_step(nei, binput, w_h):
    bp = nei.shape[0]
    grid = bp // _BR_B
    return pl.pallas_call(
        _iter_body,
        grid=(grid,),
        in_specs=[
            pl.BlockSpec((_BR_B, _HID), lambda i: (i, 0)),
            pl.BlockSpec((_BR_B, _HID), lambda i: (i, 0)),
            pl.BlockSpec((_HID, _HID), lambda i: (0, 0)),
        ],
        out_specs=pl.BlockSpec((_BR_B, _HID), lambda i: (i, 0)),
        out_shape=jax.ShapeDtypeStruct((bp, _HID), jnp.float32),
    )(nei, binput, w_h)


def _final_body(n_valid, fa_ref, nei_ref, wa_ref, wn_ref, bo_ref, out_ref):
    h = _nt(fa_ref[...], wa_ref[...]) + _nt(nei_ref[...], wn_ref[...])
    h = jnp.maximum(h + bo_ref[...], 0.0)
    row = (lax.broadcasted_iota(jnp.int32, (_BR_A, _HID), 0)
           + pl.program_id(0) * _BR_A)
    h = jnp.where(row < n_valid, h, 0.0)
    nm = _BR_A // _MLEN
    am = lax.broadcasted_iota(jnp.int32, (nm, _BR_A), 1) // _MLEN
    mm = lax.broadcasted_iota(jnp.int32, (nm, _BR_A), 0)
    pool = (am == mm).astype(jnp.float32)
    out_ref[...] = lax.dot_general(pool, h, (((1,), (0,)), ((), ())),
                                   preferred_element_type=jnp.float32)


def _final_stage(fatoms, nei_a, w_oa, w_on, b_o2d, ap):
    n, fa = fatoms.shape
    grid = ap // _BR_A
    nm = _BR_A // _MLEN
    return pl.pallas_call(
        functools.partial(_final_body, n),
        grid=(grid,),
        in_specs=[
            pl.BlockSpec((_BR_A, fa), lambda i: (i, 0)),
            pl.BlockSpec((_BR_A, _HID), lambda i: (i, 0)),
            pl.BlockSpec((_HID, fa), lambda i: (0, 0)),
            pl.BlockSpec((_HID, _HID), lambda i: (0, 0)),
            pl.BlockSpec((1, _HID), lambda i: (0, 0)),
        ],
        out_specs=pl.BlockSpec((nm, _HID), lambda i: (i, 0)),
        out_shape=jax.ShapeDtypeStruct((grid * nm, _HID), jnp.float32),
    )(fatoms, nei_a, w_oa, w_on, b_o2d)


def kernel(fatoms, fbonds, agraph, bgraph, scope, W_i, W_h, W_o, b_o):
    n_atoms, atom_fdim = fatoms.shape
    n_bonds = fbonds.shape[0]
    n_mols = scope.shape[0]

    bp = 200704
    ap = 102400

    bg_idx = _prep_idx(bgraph, bp)
    ag_idx = _prep_idx(agraph, ap)

    binput, message = _bonds_input(fbonds, W_i, bp)
    gather_b = _make_gather_sum(bp, 114)
    for _ in range(_DEPTH - 1):
        nei = gather_b(message, bg_idx)
        message = _iter_step(nei, binput, W_h)

    gather_a = _make_gather_sum(ap, 58)
    nei_a = gather_a(message, ag_idx)

    w_oa = W_o[:, :atom_fdim]
    w_on = W_o[:, atom_fdim:]
    mol_sums = _final_stage(fatoms, nei_a, w_oa, w_on,
                            b_o.reshape(1, _HID), ap)
    lengths = scope[:, 1].astype(jnp.float32)
    return mol_sums[:n_mols] / lengths[:, None]


# speedup vs baseline: 2.1899x; 1.1956x over previous
import functools

import jax
import jax.numpy as jnp
from jax import lax
from jax.experimental import pallas as pl
from jax.experimental.pallas import tpu as pltpu
from jax.experimental.pallas import tpu_sc as plsc

_HID = 128
_NW = 32
_C = 64
_DEPTH = 4
_MLEN = 25

_BR_B = 1024
_BR_A = 3200


def _make_gather_sum(bp):
    cpw = bp // _NW
    nchunk = cpw // _C
    mesh = plsc.VectorSubcoreMesh(core_axis_name="c", subcore_axis_name="s")

    @functools.partial(
        pl.kernel,
        mesh=mesh,
        out_type=jax.ShapeDtypeStruct((bp, _HID), jnp.float32),
        scratch_types=[
            pltpu.VMEM((2, 6, _C), jnp.int32),
            pltpu.VMEM((2, 6, _C, _HID), jnp.float32),
            pltpu.VMEM((2, _C, _HID), jnp.float32),
            pltpu.SemaphoreType.DMA,
            pltpu.SemaphoreType.DMA,
            pltpu.SemaphoreType.DMA,
            pltpu.SemaphoreType.DMA,
            pltpu.SemaphoreType.DMA,
            pltpu.SemaphoreType.DMA,
        ],
    )
    def gather_sum(table_hbm, idx_hbm, out_hbm, idx_v, bufs_v, out_v,
                   sem_a, sem_b, isem_a, isem_b, osem_a, osem_b):
        cid = lax.axis_index("c")
        sid = lax.axis_index("s")
        wid = sid * 2 + cid
        base = wid * cpw
        sems = (sem_a, sem_b)
        isems = (isem_a, isem_b)
        osems = (osem_a, osem_b)

        def idx_start(ci, s):
            pltpu.async_copy(idx_hbm.at[wid, ci], idx_v.at[s], isems[s])

        def idx_wait(ci, s):
            pltpu.make_async_copy(idx_hbm.at[wid, ci], idx_v.at[s],
                                  isems[s]).wait()

        def gathers(s):
            for j in range(6):
                pltpu.async_copy(table_hbm.at[idx_v.at[s, j]],
                                 bufs_v.at[s, j], sems[s])

        def drain(s):
            for j in range(6):
                pltpu.make_async_copy(table_hbm.at[idx_v.at[s, j]],
                                      bufs_v.at[s, j], sems[s]).wait()

        def out_desc(s, c):
            return pltpu.make_async_copy(
                out_v.at[s], out_hbm.at[pl.ds(base + c * _C, _C)], osems[s])

        def add(s):
            def row(r, c2):
                for l in range(8):
                    sl = pl.ds(l * 16, 16)
                    acc = bufs_v[s, 0, r, sl] + bufs_v[s, 1, r, sl]
                    acc = acc + bufs_v[s, 2, r, sl]
                    acc = acc + bufs_v[s, 3, r, sl]
                    acc = acc + bufs_v[s, 4, r, sl]
                    acc = acc + bufs_v[s, 5, r, sl]
                    out_v[s, r, sl] = acc
                return c2

            lax.fori_loop(0, _C, row, 0, unroll=2)

        for s in range(2):
            idx_start(s, s)
            idx_wait(s, s)
            gathers(s)

        def pair(p, carry):
            c0 = 2 * p
            for s in range(2):
                c = c0 + s
                drain(s)

                @pl.when(c + 2 < nchunk)
                def _():
                    idx_start(c + 2, s)

                @pl.when(c >= 2)
                def _():
                    out_desc(s, c - 2).wait()

                add(s)

                @pl.when(c + 2 < nchunk)
                def _():
                    idx_wait(c + 2, s)
                    gathers(s)

                pltpu.async_copy(out_v.at[s],
                                 out_hbm.at[pl.ds(base + c * _C, _C)],
                                 osems[s])
            return carry

        lax.fori_loop(0, nchunk // 2, pair, 0)
        for s in range(2):
            out_desc(s, nchunk - 2 + s).wait()

    return gather_sum


def _prep_idx(g, bp):
    b = g.shape[0]
    cpw = bp // _NW
    nchunk = cpw // _C
    gi = g.astype(jnp.int32)
    gi = jnp.pad(gi, ((0, bp - b), (0, 0)))
    gi = gi.reshape(_NW, nchunk, _C, 6)
    return gi.transpose(0, 1, 3, 2)


def _nt(x, w):
    return lax.dot_general(x, w, (((1,), (1,)), ((), ())),
                           preferred_element_type=jnp.float32)


def _bonds_input_body(x_ref, w_ref, bin_ref, msg_ref):
    b = _nt(x_ref[...], w_ref[...])
    bin_ref[...] = b
    msg_ref[...] = jnp.maximum(b, 0.0)


def _bonds_input(fbonds, w_i, bp):
    n, k = fbonds.shape
    grid = bp // _BR_B
    return pl.pallas_call(
        _bonds_input_body,
        grid=(grid,),
        in_specs=[
            pl.BlockSpec((_BR_B, k), lambda i: (i, 0)),
            pl.BlockSpec((_HID, k), lambda i: (0, 0)),
        ],
        out_specs=[
            pl.BlockSpec((_BR_B, _HID), lambda i: (i, 0)),
            pl.BlockSpec((_BR_B, _HID), lambda i: (i, 0)),
        ],
        out_shape=[
            jax.ShapeDtypeStruct((bp, _HID), jnp.float32),
            jax.ShapeDtypeStruct((bp, _HID), jnp.float32),
        ],
    )(fbonds, w_i)


def _iter_body(nei_ref, bin_ref, w_ref, out_ref):
    h = _nt(nei_ref[...], w_ref[...])
    out_ref[...] = jnp.maximum(bin_ref[...] + h, 0.0)


def _iter_step(nei, binput, w_h):
    bp = nei.shape[0]
    grid = bp // _BR_B
    return pl.pallas_call(
        _iter_body,
        grid=(grid,),
        in_specs=[
            pl.BlockSpec((_BR_B, _HID), lambda i: (i, 0)),
            pl.BlockSpec((_BR_B, _HID), lambda i: (i, 0)),
            pl.BlockSpec((_HID, _HID), lambda i: (0, 0)),
        ],
        out_specs=pl.BlockSpec((_BR_B, _HID), lambda i: (i, 0)),
        out_shape=jax.ShapeDtypeStruct((bp, _HID), jnp.float32),
    )(nei, binput, w_h)


def _final_body(n_valid, fa_ref, nei_ref, wa_ref, wn_ref, bo_ref, out_ref):
    h = _nt(fa_ref[...], wa_ref[...]) + _nt(nei_ref[...], wn_ref[...])
    h = jnp.maximum(h + bo_ref[...], 0.0)
    row = (lax.broadcasted_iota(jnp.int32, (_BR_A, _HID), 0)
           + pl.program_id(0) * _BR_A)
    h = jnp.where(row < n_valid, h, 0.0)
    nm = _BR_A // _MLEN
    am = lax.broadcasted_iota(jnp.int32, (nm, _BR_A), 1) // _MLEN
    mm = lax.broadcasted_iota(jnp.int32, (nm, _BR_A), 0)
    pool = (am == mm).astype(jnp.float32)
    out_ref[...] = lax.dot_general(pool, h, (((1,), (0,)), ((), ())),
                                   preferred_element_type=jnp.float32)


def _final_stage(fatoms, nei_a, w_oa, w_on, b_o2d, ap):
    n, fa = fatoms.shape
    grid = ap // _BR_A
    nm = _BR_A // _MLEN
    return pl.pallas_call(
        functools.partial(_final_body, n),
        grid=(grid,),
        in_specs=[
            pl.BlockSpec((_BR_A, fa), lambda i: (i, 0)),
            pl.BlockSpec((_BR_A, _HID), lambda i: (i, 0)),
            pl.BlockSpec((_HID, fa), lambda i: (0, 0)),
            pl.BlockSpec((_HID, _HID), lambda i: (0, 0)),
            pl.BlockSpec((1, _HID), lambda i: (0, 0)),
        ],
        out_specs=pl.BlockSpec((nm, _HID), lambda i: (i, 0)),
        out_shape=jax.ShapeDtypeStruct((grid * nm, _HID), jnp.float32),
    )(fatoms, nei_a, w_oa, w_on, b_o2d)


def kernel(fatoms, fbonds, agraph, bgraph, scope, W_i, W_h, W_o, b_o):
    n_atoms, atom_fdim = fatoms.shape
    n_bonds = fbonds.shape[0]
    n_mols = scope.shape[0]

    bp = 200704
    ap = 102400

    bg_idx = _prep_idx(bgraph, bp)
    ag_idx = _prep_idx(agraph, ap)

    binput, message = _bonds_input(fbonds, W_i, bp)
    gather_b = _make_gather_sum(bp)
    for _ in range(_DEPTH - 1):
        nei = gather_b(message, bg_idx)
        message = _iter_step(nei, binput, W_h)

    gather_a = _make_gather_sum(ap)
    nei_a = gather_a(message, ag_idx)

    w_oa = W_o[:, :atom_fdim]
    w_on = W_o[:, atom_fdim:]
    mol_sums = _final_stage(fatoms, nei_a, w_oa, w_on,
                            b_o.reshape(1, _HID), ap)
    lengths = scope[:, 1].astype(jnp.float32)
    return mol_sums[:n_mols] / lengths[:, None]


# speedup vs baseline: 2.2385x; 1.0222x over previous
import functools

import jax
import jax.numpy as jnp
from jax import lax
from jax.experimental import pallas as pl
from jax.experimental.pallas import tpu as pltpu
from jax.experimental.pallas import tpu_sc as plsc

_HID = 128
_NW = 32
_C = 64
_DEPTH = 4
_MLEN = 25

_BR_B = 1024
_BR_A = 3200


def _make_gather_sum(bp):
    cpw = bp // _NW
    nchunk = cpw // _C
    mesh = plsc.VectorSubcoreMesh(core_axis_name="c", subcore_axis_name="s")

    @functools.partial(
        pl.kernel,
        mesh=mesh,
        out_type=jax.ShapeDtypeStruct((bp, _HID), jnp.float32),
        scratch_types=[
            pltpu.VMEM((2, 6, _C), jnp.int32),
            pltpu.VMEM((2, 6, _C, _HID), jnp.float32),
            pltpu.VMEM((2, _C, _HID), jnp.float32),
            pltpu.SemaphoreType.DMA,
            pltpu.SemaphoreType.DMA,
            pltpu.SemaphoreType.DMA,
            pltpu.SemaphoreType.DMA,
            pltpu.SemaphoreType.DMA,
            pltpu.SemaphoreType.DMA,
        ],
    )
    def gather_sum(table_hbm, idx_hbm, out_hbm, idx_v, bufs_v, out_v,
                   sem_a, sem_b, isem_a, isem_b, osem_a, osem_b):
        cid = lax.axis_index("c")
        sid = lax.axis_index("s")
        wid = sid * 2 + cid
        base = wid * cpw
        sems = (sem_a, sem_b)
        isems = (isem_a, isem_b)
        osems = (osem_a, osem_b)

        def idx_start(ci, s):
            pltpu.async_copy(idx_hbm.at[wid, ci], idx_v.at[s], isems[s])

        def idx_wait(ci, s):
            pltpu.make_async_copy(idx_hbm.at[wid, ci], idx_v.at[s],
                                  isems[s]).wait()

        def gathers(s):
            for j in range(6):
                pltpu.async_copy(table_hbm.at[idx_v.at[s, j]],
                                 bufs_v.at[s, j], sems[s])

        def drain(s):
            for j in range(6):
                pltpu.make_async_copy(table_hbm.at[idx_v.at[s, j]],
                                      bufs_v.at[s, j], sems[s]).wait()

        def out_desc(s, c):
            return pltpu.make_async_copy(
                out_v.at[s], out_hbm.at[pl.ds(base + c * _C, _C)], osems[s])

        def add(s):
            def row(r, c2):
                for l in range(8):
                    sl = pl.ds(l * 16, 16)
                    z = jnp.float32(0.0)
                    acc = (jnp.maximum(bufs_v[s, 0, r, sl], z)
                           + jnp.maximum(bufs_v[s, 1, r, sl], z))
                    acc = acc + jnp.maximum(bufs_v[s, 2, r, sl], z)
                    acc = acc + jnp.maximum(bufs_v[s, 3, r, sl], z)
                    acc = acc + jnp.maximum(bufs_v[s, 4, r, sl], z)
                    acc = acc + jnp.maximum(bufs_v[s, 5, r, sl], z)
                    out_v[s, r, sl] = acc
                return c2

            lax.fori_loop(0, _C, row, 0, unroll=2)

        for s in range(2):
            idx_start(s, s)
            idx_wait(s, s)
            gathers(s)

        def pair(p, carry):
            c0 = 2 * p
            for s in range(2):
                c = c0 + s
                drain(s)

                @pl.when(c + 2 < nchunk)
                def _():
                    idx_start(c + 2, s)

                @pl.when(c >= 2)
                def _():
                    out_desc(s, c - 2).wait()

                add(s)

                @pl.when(c + 2 < nchunk)
                def _():
                    idx_wait(c + 2, s)
                    gathers(s)

                pltpu.async_copy(out_v.at[s],
                                 out_hbm.at[pl.ds(base + c * _C, _C)],
                                 osems[s])
            return carry

        lax.fori_loop(0, nchunk // 2, pair, 0)
        for s in range(2):
            out_desc(s, nchunk - 2 + s).wait()

    return gather_sum


def _prep_idx(g, bp):
    b = g.shape[0]
    cpw = bp // _NW
    nchunk = cpw // _C
    gi = g.astype(jnp.int32)
    gi = jnp.pad(gi, ((0, bp - b), (0, 0)))
    gi = gi.reshape(_NW, nchunk, _C, 6)
    return gi.transpose(0, 1, 3, 2)


def _nt(x, w):
    return lax.dot_general(x, w, (((1,), (1,)), ((), ())),
                           preferred_element_type=jnp.float32)


def _bonds_input_body(x_ref, w_ref, bin_ref):
    bin_ref[...] = _nt(x_ref[...], w_ref[...])


def _bonds_input(fbonds, w_i, bp):
    n, k = fbonds.shape
    grid = bp // _BR_B
    return pl.pallas_call(
        _bonds_input_body,
        grid=(grid,),
        in_specs=[
            pl.BlockSpec((_BR_B, k), lambda i: (i, 0)),
            pl.BlockSpec((_HID, k), lambda i: (0, 0)),
        ],
        out_specs=pl.BlockSpec((_BR_B, _HID), lambda i: (i, 0)),
        out_shape=jax.ShapeDtypeStruct((bp, _HID), jnp.float32),
    )(fbonds, w_i)


def _iter_body(nei_ref, bin_ref, w_ref, out_ref):
    out_ref[...] = bin_ref[...] + _nt(nei_ref[...], w_ref[...])


def _iter_step(nei, binput, w_h):
    bp = nei.shape[0]
    grid = bp // _BR_B
    return pl.pallas_call(
        _iter_body,
        grid=(grid,),
        in_specs=[
            pl.BlockSpec((_BR_B, _HID), lambda i: (i, 0)),
            pl.BlockSpec((_BR_B, _HID), lambda i: (i, 0)),
            pl.BlockSpec((_HID, _HID), lambda i: (0, 0)),
        ],
        out_specs=pl.BlockSpec((_BR_B, _HID), lambda i: (i, 0)),
        out_shape=jax.ShapeDtypeStruct((bp, _HID), jnp.float32),
    )(nei, binput, w_h)


def _final_body(n_valid, fa_ref, nei_ref, wa_ref, wn_ref, bo_ref, out_ref):
    h = _nt(fa_ref[...], wa_ref[...]) + _nt(nei_ref[...], wn_ref[...])
    h = jnp.maximum(h + bo_ref[...], 0.0)
    row = (lax.broadcasted_iota(jnp.int32, (_BR_A, _HID), 0)
           + pl.program_id(0) * _BR_A)
    h = jnp.where(row < n_valid, h, 0.0)
    nm = _BR_A // _MLEN
    am = lax.broadcasted_iota(jnp.int32, (nm, _BR_A), 1) // _MLEN
    mm = lax.broadcasted_iota(jnp.int32, (nm, _BR_A), 0)
    pool = (am == mm).astype(jnp.float32)
    out_ref[...] = lax.dot_general(pool, h, (((1,), (0,)), ((), ())),
                                   preferred_element_type=jnp.float32)


def _final_stage(fatoms, nei_a, w_oa, w_on, b_o2d, ap):
    n, fa = fatoms.shape
    grid = ap // _BR_A
    nm = _BR_A // _MLEN
    return pl.pallas_call(
        functools.partial(_final_body, n),
        grid=(grid,),
        in_specs=[
            pl.BlockSpec((_BR_A, fa), lambda i: (i, 0)),
            pl.BlockSpec((_BR_A, _HID), lambda i: (i, 0)),
            pl.BlockSpec((_HID, fa), lambda i: (0, 0)),
            pl.BlockSpec((_HID, _HID), lambda i: (0, 0)),
            pl.BlockSpec((1, _HID), lambda i: (0, 0)),
        ],
        out_specs=pl.BlockSpec((nm, _HID), lambda i: (i, 0)),
        out_shape=jax.ShapeDtypeStruct((grid * nm, _HID), jnp.float32),
    )(fatoms, nei_a, w_oa, w_on, b_o2d)


def kernel(fatoms, fbonds, agraph, bgraph, scope, W_i, W_h, W_o, b_o):
    n_atoms, atom_fdim = fatoms.shape
    n_bonds = fbonds.shape[0]
    n_mols = scope.shape[0]

    bp = 200704
    ap = 102400

    bg_idx = _prep_idx(bgraph, bp)
    ag_idx = _prep_idx(agraph, ap)

    binput = _bonds_input(fbonds, W_i, bp)
    gather_b = _make_gather_sum(bp)
    z = binput
    for _ in range(_DEPTH - 1):
        nei = gather_b(z, bg_idx)
        z = _iter_step(nei, binput, W_h)

    gather_a = _make_gather_sum(ap)
    nei_a = gather_a(z, ag_idx)

    w_oa = W_o[:, :atom_fdim]
    w_on = W_o[:, atom_fdim:]
    mol_sums = _final_stage(fatoms, nei_a, w_oa, w_on,
                            b_o.reshape(1, _HID), ap)
    lengths = scope[:, 1].astype(jnp.float32)
    return mol_sums[:n_mols] / lengths[:, None]
